# baseline (speedup 1.0000x reference)
import jax
import jax.numpy as jnp
from jax.experimental import pallas as pl
from jax.experimental.pallas import tpu as pltpu

_CN = 96


def _body(mod_ref, x_ref, cb_ref, out_ref, codes_scr, aug_scr):
    nb = x_ref.shape[0]
    cn = x_ref.shape[1]
    m = cb_ref.shape[0]
    k = cb_ref.shape[1]

    d = cb_ref.shape[2]

    @pl.when(pl.program_id(0) == 0)
    def _init_c2():
        ones = jnp.ones((1, d), jnp.float32)
        for mm in range(m):
            cbm = cb_ref[mm]
            aug_scr[mm] = jax.lax.dot_general(
                cbm * cbm, ones, (((1,), (1,)), ((), ())),
                preferred_element_type=jnp.float32)

    for b in range(nb):
        cb = cb_ref[mod_ref[b]]
        xm2 = x_ref[b] * -2.0
        cross_t = jax.lax.dot_general(
            cb, xm2, (((1,), (1,)), ((), ())),
            preferred_element_type=jnp.float32)
        dists = aug_scr[mod_ref[b]] + cross_t
        code = jnp.argmin(dists, axis=0)
        codes_scr[b:b + 1, :] = code[None, :]

    iota = jax.lax.broadcasted_iota(jnp.int32, (nb, k), 1)
    for j in range(cn):
        cvec = codes_scr[:, j:j + 1]
        out_ref[:, j * k:(j + 1) * k] = (iota == cvec).astype(jnp.int32)


def kernel(input, mod, codebooks):
    b, n, d = input.shape
    m, k, _ = codebooks.shape
    nsteps = pl.cdiv(n, _CN)

    grid_spec = pltpu.PrefetchScalarGridSpec(
        num_scalar_prefetch=1,
        grid=(nsteps,),
        in_specs=[
            pl.BlockSpec((b, _CN, d), lambda s, mod_ref: (0, s, 0)),
            pl.BlockSpec((m, k, d), lambda s, mod_ref: (0, 0, 0)),
        ],
        out_specs=pl.BlockSpec((b, _CN * k), lambda s, mod_ref: (0, s)),
        scratch_shapes=[
            pltpu.VMEM((b, _CN), jnp.int32),
            pltpu.VMEM((m, k, 1), jnp.float32),
        ],
    )
    out = pl.pallas_call(
        _body,
        grid_spec=grid_spec,
        out_shape=jax.ShapeDtypeStruct((b, n * k), jnp.int32),
    )(mod, input, codebooks)
    return out


# speedup vs baseline: 1.2098x; 1.2098x over previous
import jax
import jax.numpy as jnp
from jax.experimental import pallas as pl
from jax.experimental.pallas import tpu as pltpu

_CN = 128


def _body(mod_ref, x_ref, cb_ref, out_ref, codes_scr, aug_scr):
    nb = x_ref.shape[0]
    cn = x_ref.shape[1]
    m = cb_ref.shape[0]
    k = cb_ref.shape[1]

    d = cb_ref.shape[2]

    @pl.when(pl.program_id(0) == 0)
    def _init_c2():
        ones = jnp.ones((1, d), jnp.float32)
        for mm in range(m):
            cbm = cb_ref[mm]
            c2col = jax.lax.dot_general(
                cbm * cbm, ones, (((1,), (1,)), ((), ())),
                preferred_element_type=jnp.float32)
            aug_scr[mm] = jnp.broadcast_to(c2col, (k, cn))

    for b in range(nb):
        cb = cb_ref[mod_ref[b]]
        xm2 = x_ref[b] * -2.0
        cross_t = jax.lax.dot_general(
            cb, xm2, (((1,), (1,)), ((), ())),
            preferred_element_type=jnp.float32)
        dists = aug_scr[mod_ref[b]] + cross_t
        code = jnp.argmin(dists, axis=0)
        codes_scr[b:b + 1, :] = code[None, :]

    iota = jax.lax.broadcasted_iota(jnp.int32, (nb, k), 1)
    for j in range(cn):
        cvec = codes_scr[:, j:j + 1]
        out_ref[:, j * k:(j + 1) * k] = (iota == cvec).astype(jnp.int32)


def kernel(input, mod, codebooks):
    b, n, d = input.shape
    m, k, _ = codebooks.shape
    nsteps = pl.cdiv(n, _CN)

    grid_spec = pltpu.PrefetchScalarGridSpec(
        num_scalar_prefetch=1,
        grid=(nsteps,),
        in_specs=[
            pl.BlockSpec((b, _CN, d), lambda s, mod_ref: (0, s, 0)),
            pl.BlockSpec((m, k, d), lambda s, mod_ref: (0, 0, 0)),
        ],
        out_specs=pl.BlockSpec((b, _CN * k), lambda s, mod_ref: (0, s)),
        scratch_shapes=[
            pltpu.VMEM((b, _CN), jnp.int32),
            pltpu.VMEM((m, k, _CN), jnp.float32),
        ],
    )
    out = pl.pallas_call(
        _body,
        grid_spec=grid_spec,
        out_shape=jax.ShapeDtypeStruct((b, n * k), jnp.int32),
    )(mod, input, codebooks)
    return out
